# initial kernel scaffold (unmeasured)
import jax
import jax.numpy as jnp
import numpy as np
from jax import lax
from jax.experimental import pallas as pl
from jax.experimental.pallas import tpu as pltpu

N_DEV = 16
M, N = 8192, 4096
HALF = M // 2
CHUNK = HALF // N_DEV
N_STEP = N_DEV - 1


def _ring_perm() -> list[int]:
    try:
        devs = jax.devices()
        coords = sorted(
            tuple(d.coords)
            for d in devs
            if getattr(d, "core_on_chip", 1) == 1
        )
        if len(coords) != N_DEV or coords != [
            (x, y, z) for z in range(4) for y in range(2) for x in range(2)
        ] and set(coords) != {(x, y, z) for x in range(2) for y in range(2) for z in range(4)}:
            raise ValueError("unexpected topology")
        face = [(0, 0), (1, 0), (1, 1), (0, 1)]
        log_of = {}
        for z in range(4):
            for fi, (x, y) in enumerate(face):
                log_of[(x, y, z)] = 4 * z + fi
        cycle = []
        for fi, (x, y) in enumerate(face):
            zs = range(4) if fi % 2 == 0 else range(3, -1, -1)
            cycle.extend((x, y, z) for z in zs)
        return [log_of[c] for c in cycle]
    except Exception:
        return list(range(N_DEV))


_PERM = _ring_perm()
_RANK = [0] * N_DEV
for _pos, _l in enumerate(_PERM):
    _RANK[_l] = _pos


def kernel(x, w_mat):
    perm = jnp.asarray(_PERM, dtype=jnp.int32)
    rank_arr = jnp.asarray(_RANK, dtype=jnp.int32)
    me = lax.axis_index("i")
    rank = rank_arr[me]
    right = perm[(rank + 1) % N_DEV]
    left = perm[(rank - 1) % N_DEV]
    meta = jnp.stack(
        [rank, (N_DEV - rank) % N_DEV, right, left]
    ).astype(jnp.int32)

    def body(meta_ref, x_ref, w_ref, out_ref, send_v, recv_v, out_v,
             rs_hbm, ag_hbm, rs_sem, ag_sem, rs_send, ag_send,
             copy_sem, out_sem):
        ranks = (meta_ref[0], meta_ref[1])
        peers = (meta_ref[2], meta_ref[3])

        def cidx(d, s_off):
            return lax.rem(ranks[d] - s_off + 2 * N_DEV, N_DEV)

        def rows(d, c):
            return d * HALF + c * CHUNK

        def partial_chunk(d, c):
            xs = x_ref[pl.ds(rows(d, c), CHUNK), :]
            return jnp.dot(xs, w_ref[:, :], preferred_element_type=jnp.float32)

        for d in (0, 1):
            send_v[d, 0] = partial_chunk(d, cidx(d, 0)).astype(jnp.bfloat16)

        bar = pltpu.get_barrier_semaphore()
        for d in (0, 1):
            pl.semaphore_signal(
                bar, inc=1, device_id=peers[d],
                device_id_type=pl.DeviceIdType.LOGICAL,
            )
        pl.semaphore_wait(bar, 2)

        prev = [None, None]
        for s in range(N_STEP):
            cur = []
            for d in (0, 1):
                r = pltpu.make_async_remote_copy(
                    src_ref=send_v.at[d, s % 2],
                    dst_ref=rs_hbm.at[d, s],
                    send_sem=rs_send.at[d, s % 2],
                    recv_sem=rs_sem.at[d, s],
                    device_id=peers[d],
                    device_id_type=pl.DeviceIdType.LOGICAL,
                )
                r.start()
                cur.append(r)
            for d in (0, 1):
                c = cidx(d, s + 1)
                acc = partial_chunk(d, c)
                cur[d].wait_recv()
                cp = pltpu.make_async_copy(
                    rs_hbm.at[d, s], recv_v.at[d], copy_sem.at[d]
                )
                cp.start()
                cp.wait()
                tot = acc + recv_v[d].astype(jnp.float32)
                if s < N_STEP - 1:
                    if prev[d] is not None:
                        prev[d].wait_send()
                    send_v[d, (s + 1) % 2] = tot.astype(jnp.bfloat16)
                else:
                    out_v[d] = tot
                    ocp = pltpu.make_async_copy(
                        out_v.at[d],
                        out_ref.at[pl.ds(rows(d, c), CHUNK), :],
                        out_sem.at[d],
                    )
                    ocp.start()
                    prev[d].wait_send()
                    send_v[d, 1] = tot.astype(jnp.bfloat16)
                    out_pending = ocp
            prev = cur

        out_pend = [None, None]
        for d in (0, 1):
            pass

        ag_prev = [[None, None], [None, None]]
        out_dma = [None, None]
        cur = [None, None]
        for s in range(N_STEP):
            for d in (0, 1):
                src = send_v.at[d, 1] if s == 0 else ag_hbm.at[d, s - 1]
                if ag_prev[s % 2][d] is not None:
                    ag_prev[s % 2][d].wait_send()
                r = pltpu.make_async_remote_copy(
                    src_ref=src,
                    dst_ref=ag_hbm.at[d, s],
                    send_sem=ag_send.at[d, s % 2],
                    recv_sem=ag_sem.at[d, s],
                    device_id=peers[d],
                    device_id_type=pl.DeviceIdType.LOGICAL,
                )
                r.start()
                ag_prev[s % 2][d] = r
            if s >= 1:
                for d in (0, 1):
                    c = cidx(d, s - 1)
                    cp = pltpu.make_async_copy(
                        ag_hbm.at[d, s - 1], recv_v.at[d], copy_sem.at[d]
                    )
                    cp.start()
                    cp.wait()
                    if out_dma[d] is not None:
                        out_dma[d].wait()
                    out_v[d] = recv_v[d].astype(jnp.float32)
                    ocp = pltpu.make_async_copy(
                        out_v.at[d],
                        out_ref.at[pl.ds(rows(d, c), CHUNK), :],
                        out_sem.at[d],
                    )
                    ocp.start()
                    out_dma[d] = ocp
            for d in (0, 1):
                ag_prev[s % 2][d].wait_recv()
                cur[d] = ag_prev[s % 2][d]

        for d in (0, 1):
            c = cidx(d, N_STEP - 1)
            cp = pltpu.make_async_copy(
                ag_hbm.at[d, N_STEP - 1], recv_v.at[d], copy_sem.at[d]
            )
            cp.start()
            cp.wait()
            if out_dma[d] is not None:
                out_dma[d].wait()
            out_v[d] = recv_v[d].astype(jnp.float32)
            ocp = pltpu.make_async_copy(
                out_v.at[d],
                out_ref.at[pl.ds(rows(d, c), CHUNK), :],
                out_sem.at[d],
            )
            ocp.start()
            ocp.wait()
        for d in (0, 1):
            prev[d].wait_send()
            for par in (0, 1):
                if ag_prev[par][d] is not None:
                    ag_prev[par][d].wait_send()

    scratch = [
        pltpu.VMEM((2, 2, CHUNK, N), jnp.bfloat16),
        pltpu.VMEM((2, CHUNK, N), jnp.bfloat16),
        pltpu.VMEM((2, CHUNK, N), jnp.float32),
        pltpu.MemorySpace.HBM((2, N_STEP, CHUNK, N), jnp.bfloat16),
        pltpu.MemorySpace.HBM((2, N_STEP, CHUNK, N), jnp.bfloat16),
        pltpu.SemaphoreType.DMA((2, N_STEP)),
        pltpu.SemaphoreType.DMA((2, N_STEP)),
        pltpu.SemaphoreType.DMA((2, 2)),
        pltpu.SemaphoreType.DMA((2, 2)),
        pltpu.SemaphoreType.DMA((2,)),
        pltpu.SemaphoreType.DMA((2,)),
    ]

    return pl.pallas_call(
        body,
        out_shape=jax.ShapeDtypeStruct((M, N), jnp.float32),
        in_specs=[
            pl.BlockSpec(memory_space=pltpu.MemorySpace.SMEM),
            pl.BlockSpec(memory_space=pltpu.MemorySpace.VMEM),
            pl.BlockSpec(memory_space=pltpu.MemorySpace.VMEM),
        ],
        out_specs=pl.BlockSpec(memory_space=pltpu.MemorySpace.HBM),
        scratch_shapes=scratch,
        compiler_params=pltpu.CompilerParams(collective_id=0),
    )(meta, x, w_mat)


# baseline (device time: 967103 ns/iter reference)
import jax
import jax.numpy as jnp
import numpy as np
from jax import lax
from jax.experimental import pallas as pl
from jax.experimental.pallas import tpu as pltpu

N_DEV = 16
M, N = 8192, 4096
HALF = M // 2
CHUNK = HALF // N_DEV
N_STEP = N_DEV - 1


def _ring_perm() -> list[int]:
    try:
        devs = jax.devices()
        coords = sorted(
            tuple(d.coords)
            for d in devs
            if getattr(d, "core_on_chip", 1) == 1
        )
        if len(coords) != N_DEV or coords != [
            (x, y, z) for z in range(4) for y in range(2) for x in range(2)
        ] and set(coords) != {(x, y, z) for x in range(2) for y in range(2) for z in range(4)}:
            raise ValueError("unexpected topology")
        face = [(0, 0), (1, 0), (1, 1), (0, 1)]
        log_of = {}
        for z in range(4):
            for fi, (x, y) in enumerate(face):
                log_of[(x, y, z)] = 4 * z + fi
        cycle = []
        for fi, (x, y) in enumerate(face):
            zs = range(4) if fi % 2 == 0 else range(3, -1, -1)
            cycle.extend((x, y, z) for z in zs)
        return [log_of[c] for c in cycle]
    except Exception:
        return list(range(N_DEV))


_PERM = _ring_perm()
_RANK = [0] * N_DEV
for _pos, _l in enumerate(_PERM):
    _RANK[_l] = _pos


def kernel(x, w_mat):
    perm = jnp.asarray(_PERM, dtype=jnp.int32)
    rank_arr = jnp.asarray(_RANK, dtype=jnp.int32)
    me = lax.axis_index("i")
    rank = rank_arr[me]
    right = perm[(rank + 1) % N_DEV]
    left = perm[(rank - 1) % N_DEV]
    meta = jnp.stack(
        [rank, (N_DEV - rank) % N_DEV, right, left]
    ).astype(jnp.int32)

    def body(meta_ref, x_ref, w_ref, out_ref, rs_hbm, ag_hbm,
             send_v, recv_v, out_v, rs_sem, ag_sem, rs_send, ag_send,
             copy_sem, out_sem):
        ranks = (meta_ref[0], meta_ref[1])
        peers = (meta_ref[2], meta_ref[3])

        def cidx(d, s_off):
            return lax.rem(ranks[d] - s_off + 2 * N_DEV, N_DEV)

        def rows(d, c):
            return d * HALF + c * CHUNK

        def partial_chunk(d, c):
            xs = x_ref[pl.ds(rows(d, c), CHUNK), :]
            return jnp.dot(xs, w_ref[:, :], preferred_element_type=jnp.float32)

        for d in (0, 1):
            send_v[d, 0] = partial_chunk(d, cidx(d, 0)).astype(jnp.bfloat16)

        bar = pltpu.get_barrier_semaphore()
        for d in (0, 1):
            pl.semaphore_signal(
                bar, inc=1, device_id=peers[d],
                device_id_type=pl.DeviceIdType.LOGICAL,
            )
        pl.semaphore_wait(bar, 2)

        prev = [None, None]
        out_dma = [None, None]
        for s in range(N_STEP):
            cur = []
            for d in (0, 1):
                r = pltpu.make_async_remote_copy(
                    src_ref=send_v.at[d, s % 2],
                    dst_ref=rs_hbm.at[d, s],
                    send_sem=rs_send.at[d, s % 2],
                    recv_sem=rs_sem.at[d, s],
                    device_id=peers[d],
                    device_id_type=pl.DeviceIdType.LOGICAL,
                )
                r.start()
                cur.append(r)
            for d in (0, 1):
                c = cidx(d, s + 1)
                acc = partial_chunk(d, c)
                cur[d].wait_recv()
                cp = pltpu.make_async_copy(
                    rs_hbm.at[d, s], recv_v.at[d], copy_sem.at[d]
                )
                cp.start()
                cp.wait()
                tot = acc + recv_v[d].astype(jnp.float32)
                if s < N_STEP - 1:
                    if prev[d] is not None:
                        prev[d].wait_send()
                    send_v[d, (s + 1) % 2] = tot.astype(jnp.bfloat16)
                else:
                    out_v[d] = tot
                    ocp = pltpu.make_async_copy(
                        out_v.at[d],
                        out_ref.at[pl.ds(rows(d, c), CHUNK), :],
                        out_sem.at[d],
                    )
                    ocp.start()
                    out_dma[d] = ocp
                    prev[d].wait_send()
                    send_v[d, 1] = tot.astype(jnp.bfloat16)
            prev = cur

        ag_prev = [[None, None], [None, None]]
        cur = [None, None]
        for s in range(N_STEP):
            for d in (0, 1):
                src = send_v.at[d, 1] if s == 0 else ag_hbm.at[d, s - 1]
                if ag_prev[s % 2][d] is not None:
                    ag_prev[s % 2][d].wait_send()
                r = pltpu.make_async_remote_copy(
                    src_ref=src,
                    dst_ref=ag_hbm.at[d, s],
                    send_sem=ag_send.at[d, s % 2],
                    recv_sem=ag_sem.at[d, s],
                    device_id=peers[d],
                    device_id_type=pl.DeviceIdType.LOGICAL,
                )
                r.start()
                ag_prev[s % 2][d] = r
            if s >= 1:
                for d in (0, 1):
                    c = cidx(d, s - 1)
                    cp = pltpu.make_async_copy(
                        ag_hbm.at[d, s - 1], recv_v.at[d], copy_sem.at[d]
                    )
                    cp.start()
                    cp.wait()
                    if out_dma[d] is not None:
                        out_dma[d].wait()
                    out_v[d] = recv_v[d].astype(jnp.float32)
                    ocp = pltpu.make_async_copy(
                        out_v.at[d],
                        out_ref.at[pl.ds(rows(d, c), CHUNK), :],
                        out_sem.at[d],
                    )
                    ocp.start()
                    out_dma[d] = ocp
            for d in (0, 1):
                ag_prev[s % 2][d].wait_recv()
                cur[d] = ag_prev[s % 2][d]

        for d in (0, 1):
            c = cidx(d, N_STEP - 1)
            cp = pltpu.make_async_copy(
                ag_hbm.at[d, N_STEP - 1], recv_v.at[d], copy_sem.at[d]
            )
            cp.start()
            cp.wait()
            if out_dma[d] is not None:
                out_dma[d].wait()
            out_v[d] = recv_v[d].astype(jnp.float32)
            ocp = pltpu.make_async_copy(
                out_v.at[d],
                out_ref.at[pl.ds(rows(d, c), CHUNK), :],
                out_sem.at[d],
            )
            ocp.start()
            ocp.wait()
        for d in (0, 1):
            prev[d].wait_send()
            for par in (0, 1):
                if ag_prev[par][d] is not None:
                    ag_prev[par][d].wait_send()

    scratch = [
        pltpu.VMEM((2, 2, CHUNK, N), jnp.bfloat16),
        pltpu.VMEM((2, CHUNK, N), jnp.bfloat16),
        pltpu.VMEM((2, CHUNK, N), jnp.float32),
        pltpu.SemaphoreType.DMA((2, N_STEP)),
        pltpu.SemaphoreType.DMA((2, N_STEP)),
        pltpu.SemaphoreType.DMA((2, 2)),
        pltpu.SemaphoreType.DMA((2, 2)),
        pltpu.SemaphoreType.DMA((2,)),
        pltpu.SemaphoreType.DMA((2,)),
    ]

    comm_shape = jax.ShapeDtypeStruct((2, N_STEP, CHUNK, N), jnp.bfloat16)
    out, _, _ = pl.pallas_call(
        body,
        out_shape=[
            jax.ShapeDtypeStruct((M, N), jnp.float32),
            comm_shape,
            comm_shape,
        ],
        in_specs=[
            pl.BlockSpec(memory_space=pltpu.MemorySpace.SMEM),
            pl.BlockSpec(memory_space=pltpu.MemorySpace.VMEM),
            pl.BlockSpec(memory_space=pltpu.MemorySpace.VMEM),
        ],
        out_specs=[
            pl.BlockSpec(memory_space=pltpu.MemorySpace.HBM),
            pl.BlockSpec(memory_space=pltpu.MemorySpace.HBM),
            pl.BlockSpec(memory_space=pltpu.MemorySpace.HBM),
        ],
        scratch_shapes=scratch,
        compiler_params=pltpu.CompilerParams(
            collective_id=0,
            vmem_limit_bytes=60 * 1024 * 1024,
        ),
    )(meta, x.astype(jnp.bfloat16), w_mat.astype(jnp.bfloat16))
    return out
